# baseline (speedup 1.0000x reference)
import functools

import jax
import jax.numpy as jnp
from jax import lax
from jax.experimental import pallas as pl
from jax.experimental.pallas import tpu as pltpu
from jax.experimental.pallas import tpu_sc as plsc

D = 64
NC, NS = 2, 16
NW = NC * NS
BLK = 128
L = 16


@functools.partial(jax.jit, static_argnums=(2, 3))
def _gather_t(table, x_t, seq_len, batch):
    assert batch // NW == BLK

    mesh = plsc.VectorSubcoreMesh(
        core_axis_name="c", subcore_axis_name="s",
        num_cores=NC, num_subcores=NS)

    @functools.partial(
        pl.kernel,
        mesh=mesh,
        compiler_params=pltpu.CompilerParams(
            use_tc_tiling_on_sc=False, needs_layout_passes=False),
        out_type=jax.ShapeDtypeStruct((seq_len, D, batch), jnp.float32),
        scratch_types=[
            pltpu.VMEM((seq_len, BLK), jnp.int32),
            pltpu.VMEM((2, BLK, D), jnp.float32),
            pltpu.VMEM((2, D, BLK), jnp.float32),
            pltpu.SemaphoreType.DMA,
            pltpu.SemaphoreType.DMA,
        ],
    )
    def body(table_hbm, xt_hbm, out_hbm, idx_v, rows_v, trans_v, gsem, osem):
        wid = lax.axis_index("s") * NC + lax.axis_index("c")
        col0 = wid * BLK

        pltpu.sync_copy(xt_hbm.at[:, pl.ds(col0, BLK)], idx_v)

        def fire_gather(s, b):
            pltpu.async_copy(
                table_hbm.at[idx_v.at[s]], rows_v.at[b], gsem)

        def wait_gather(s, b):
            pltpu.make_async_copy(
                table_hbm.at[idx_v.at[s]], rows_v.at[b], gsem).wait()

        def fire_store(s, b):
            pltpu.async_copy(
                trans_v.at[b], out_hbm.at[s, :, pl.ds(col0, BLK)], osem)

        def wait_store(s, b):
            pltpu.make_async_copy(
                trans_v.at[b], out_hbm.at[s, :, pl.ds(col0, BLK)], osem).wait()

        riota = lax.iota(jnp.int32, L)
        perms = [lax.rem(riota + jnp.int32(k), jnp.int32(L)) for k in range(L)]

        def transpose_block(b):
            @plsc.parallel_loop(0, (D // L) * (BLK // L), unroll=2)
            def _(t):
                di = lax.div(t, jnp.int32(BLK // L)) * L
                ri = lax.rem(t, jnp.int32(BLK // L)) * L
                rvec = riota + ri
                dvec0 = riota * jnp.int32(0) + di
                for k in range(L):
                    cvec = perms[k] + dvec0
                    v = plsc.load_gather(rows_v.at[b], [rvec, cvec])
                    plsc.store_scatter(
                        trans_v.at[b], [cvec, rvec], v)

        fire_gather(0, 0)

        @pl.loop(0, seq_len, step=2)
        def _(s2):
            for b in range(2):
                s = s2 + b
                nb = 1 - b

                @pl.when(s + 1 < seq_len)
                def _():
                    fire_gather(s + 1, nb)

                wait_gather(s, b)

                @pl.when(s >= 2)
                def _():
                    wait_store(s - 2, b)

                transpose_block(b)
                fire_store(s, b)

        wait_store(seq_len - 2, 0)
        wait_store(seq_len - 1, 1)

    return body(table, x_t)


def kernel(x, embedding_table, train):
    b, s = x.shape
    out_t = _gather_t(embedding_table, x.T, s, b)
    return out_t.transpose(2, 0, 1)


# speedup vs baseline: 1.6790x; 1.6790x over previous
import functools

import jax
import jax.numpy as jnp
from jax import lax
from jax.experimental import pallas as pl
from jax.experimental.pallas import tpu as pltpu
from jax.experimental.pallas import tpu_sc as plsc

D = 64
NC, NS = 2, 16
NW = NC * NS
BLK = 128
L = 16


_PACK_BW = 2048
_PACK_LOG = _PACK_BW.bit_length() - 1


def _pack_block(t_ref, o_ref):
    t = t_ref[...]
    o_ref[:, :D] = t[:, :_PACK_BW].T
    o_ref[:, D:] = t[:, _PACK_BW:].T


@jax.jit
def _pack(table_t):
    dd, v = table_t.shape
    grid = (v + 2 * _PACK_BW - 1) // (2 * _PACK_BW)
    return pl.pallas_call(
        _pack_block,
        grid=(grid,),
        in_specs=[pl.BlockSpec((dd, 2 * _PACK_BW), lambda i: (0, i))],
        out_specs=pl.BlockSpec((_PACK_BW, 2 * dd), lambda i: (i, 0)),
        out_shape=jax.ShapeDtypeStruct((grid * _PACK_BW, 2 * dd), jnp.float32),
    )(table_t)


@functools.partial(jax.jit, static_argnums=(2, 3))
def _gather_t(table2, x_t, seq_len, batch):
    assert batch // NW == BLK

    mesh = plsc.VectorSubcoreMesh(
        core_axis_name="c", subcore_axis_name="s",
        num_cores=NC, num_subcores=NS)

    @functools.partial(
        pl.kernel,
        mesh=mesh,
        compiler_params=pltpu.CompilerParams(
            use_tc_tiling_on_sc=True, needs_layout_passes=False),
        out_type=jax.ShapeDtypeStruct((seq_len, D, batch), jnp.float32),
        scratch_types=[
            pltpu.VMEM((seq_len, BLK), jnp.int32),
            pltpu.VMEM((seq_len, BLK), jnp.int32),
            pltpu.VMEM((2, BLK, BLK), jnp.float32),
            pltpu.VMEM((2, D, BLK), jnp.float32),
            pltpu.SemaphoreType.DMA,
            pltpu.SemaphoreType.DMA,
        ],
    )
    def body(table_hbm, xt_hbm, out_hbm, idx_v, idx2_v, rows_v, trans_v, gsem, osem):
        wid = lax.axis_index("s") * NC + lax.axis_index("c")
        col0 = wid * BLK

        pltpu.sync_copy(xt_hbm.at[:, pl.ds(col0, BLK)], idx_v)

        @plsc.parallel_loop(0, seq_len, unroll=4)
        def _(si):
            for g in range(BLK // L):
                v = idx_v[si, pl.ds(g * L, L)]
                idx2_v[si, pl.ds(g * L, L)] = lax.bitwise_or(
                    lax.shift_left(
                        lax.shift_right_logical(
                            v, jnp.int32(_PACK_LOG + 1)),
                        jnp.int32(_PACK_LOG)),
                    lax.bitwise_and(v, jnp.int32(_PACK_BW - 1)))

        def fire_gather(s, b):
            pltpu.async_copy(
                table_hbm.at[idx2_v.at[s]], rows_v.at[b], gsem)

        def wait_gather(s, b):
            pltpu.make_async_copy(
                table_hbm.at[idx2_v.at[s]], rows_v.at[b], gsem).wait()

        def fire_store(s, b):
            pltpu.async_copy(
                trans_v.at[b], out_hbm.at[s, :, pl.ds(col0, BLK)], osem)

        def wait_store(s, b):
            pltpu.make_async_copy(
                trans_v.at[b], out_hbm.at[s, :, pl.ds(col0, BLK)], osem).wait()

        riota = lax.iota(jnp.int32, L)
        perms = [lax.rem(riota + jnp.int32(k), jnp.int32(L)) for k in range(L)]

        def transpose_block(s, b):
            @plsc.parallel_loop(0, (D // L) * (BLK // L), unroll=2)
            def _(t):
                di = lax.div(t, jnp.int32(BLK // L)) * L
                ri = lax.rem(t, jnp.int32(BLK // L)) * L
                rvec = riota + ri
                pvec = lax.shift_left(
                    lax.bitwise_and(
                        lax.shift_right_logical(
                            plsc.load_gather(idx_v.at[s], [rvec]),
                            jnp.int32(_PACK_LOG)),
                        jnp.int32(1)),
                    jnp.int32(6))
                dvec0 = pvec + di
                svec0 = riota * jnp.int32(0) + di
                for k in range(L):
                    cvec = perms[k] + dvec0
                    v = plsc.load_gather(rows_v.at[b], [rvec, cvec])
                    plsc.store_scatter(
                        trans_v.at[b], [perms[k] + svec0, rvec], v)

        fire_gather(0, 0)

        @pl.loop(0, seq_len, step=2)
        def _(s2):
            for b in range(2):
                s = s2 + b
                nb = 1 - b

                @pl.when(s + 1 < seq_len)
                def _():
                    fire_gather(s + 1, nb)

                wait_gather(s, b)

                @pl.when(s >= 2)
                def _():
                    wait_store(s - 2, b)

                transpose_block(s, b)
                fire_store(s, b)

        wait_store(seq_len - 2, 0)
        wait_store(seq_len - 1, 1)

    return body(table2, x_t)


def kernel(x, embedding_table, train):
    b, s = x.shape
    table2 = _pack(embedding_table.T)
    out_t = _gather_t(table2, x.T, s, b)
    return out_t.transpose(2, 0, 1)


# speedup vs baseline: 1.8986x; 1.1308x over previous
import functools

import jax
import jax.numpy as jnp
from jax import lax
from jax.experimental import pallas as pl
from jax.experimental.pallas import tpu as pltpu
from jax.experimental.pallas import tpu_sc as plsc

D = 64
NC, NS = 2, 16
NW = NC * NS
BLK = 128
L = 16


_PACK_BW = 4096
_PACK_LOG = _PACK_BW.bit_length() - 1


def _pack_block(t_ref, o_ref):
    t = t_ref[...]
    o_ref[:, :D] = t[:, :_PACK_BW].T
    o_ref[:, D:] = t[:, _PACK_BW:].T


@jax.jit
def _pack(table_t):
    dd, v = table_t.shape
    grid = (v + 2 * _PACK_BW - 1) // (2 * _PACK_BW)
    return pl.pallas_call(
        _pack_block,
        grid=(grid,),
        in_specs=[pl.BlockSpec((dd, 2 * _PACK_BW), lambda i: (0, i))],
        out_specs=pl.BlockSpec((_PACK_BW, 2 * dd), lambda i: (i, 0)),
        out_shape=jax.ShapeDtypeStruct((grid * _PACK_BW, 2 * dd), jnp.float32),
    )(table_t)


@functools.partial(jax.jit, static_argnums=(2, 3))
def _gather_t(table2, x_t, seq_len, batch):
    assert batch // NW == BLK

    mesh = plsc.VectorSubcoreMesh(
        core_axis_name="c", subcore_axis_name="s",
        num_cores=NC, num_subcores=NS)

    @functools.partial(
        pl.kernel,
        mesh=mesh,
        compiler_params=pltpu.CompilerParams(
            use_tc_tiling_on_sc=True, needs_layout_passes=False),
        out_type=jax.ShapeDtypeStruct((seq_len, D, batch), jnp.float32),
        scratch_types=[
            pltpu.VMEM((seq_len, BLK), jnp.int32),
            pltpu.VMEM((seq_len, BLK), jnp.int32),
            pltpu.VMEM((2, BLK, BLK), jnp.float32),
            pltpu.VMEM((2, D, BLK), jnp.float32),
            pltpu.SemaphoreType.DMA,
            pltpu.SemaphoreType.DMA,
        ],
    )
    def body(table_hbm, xt_hbm, out_hbm, idx_v, idx2_v, rows_v, trans_v, gsem, osem):
        wid = lax.axis_index("s") * NC + lax.axis_index("c")
        col0 = wid * BLK

        pltpu.sync_copy(xt_hbm.at[:, pl.ds(col0, BLK)], idx_v)

        @plsc.parallel_loop(0, seq_len, unroll=4)
        def _(si):
            for g in range(BLK // L):
                v = idx_v[si, pl.ds(g * L, L)]
                idx2_v[si, pl.ds(g * L, L)] = lax.bitwise_or(
                    lax.shift_left(
                        lax.shift_right_logical(
                            v, jnp.int32(_PACK_LOG + 1)),
                        jnp.int32(_PACK_LOG)),
                    lax.bitwise_and(v, jnp.int32(_PACK_BW - 1)))

        def fire_gather(s, b):
            pltpu.async_copy(
                table_hbm.at[idx2_v.at[s]], rows_v.at[b], gsem)

        def wait_gather(s, b):
            pltpu.make_async_copy(
                table_hbm.at[idx2_v.at[s]], rows_v.at[b], gsem).wait()

        def fire_store(s, b):
            pltpu.async_copy(
                trans_v.at[b], out_hbm.at[s, :, pl.ds(col0, BLK)], osem)

        def wait_store(s, b):
            pltpu.make_async_copy(
                trans_v.at[b], out_hbm.at[s, :, pl.ds(col0, BLK)], osem).wait()

        riota = lax.iota(jnp.int32, L)
        perms = [lax.rem(riota + jnp.int32(k), jnp.int32(L)) for k in range(L)]

        def transpose_block(s, b):
            @plsc.parallel_loop(0, (D // L) * (BLK // L), unroll=2)
            def _(t):
                di = lax.div(t, jnp.int32(BLK // L)) * L
                ri = lax.rem(t, jnp.int32(BLK // L)) * L
                rvec = riota + ri
                pvec = lax.shift_left(
                    lax.bitwise_and(
                        lax.shift_right_logical(
                            plsc.load_gather(idx_v.at[s], [rvec]),
                            jnp.int32(_PACK_LOG)),
                        jnp.int32(1)),
                    jnp.int32(6))
                dvec0 = pvec + di
                svec0 = riota * jnp.int32(0) + di
                for k in range(L):
                    cvec = perms[k] + dvec0
                    v = plsc.load_gather(rows_v.at[b], [rvec, cvec])
                    plsc.store_scatter(
                        trans_v.at[b], [perms[k] + svec0, rvec], v)

        fire_gather(0, 0)

        @pl.loop(0, seq_len, step=2)
        def _(s2):
            for b in range(2):
                s = s2 + b
                nb = 1 - b

                @pl.when(s + 1 < seq_len)
                def _():
                    fire_gather(s + 1, nb)

                wait_gather(s, b)

                @pl.when(s >= 2)
                def _():
                    wait_store(s - 2, b)

                transpose_block(s, b)
                fire_store(s, b)

        wait_store(seq_len - 2, 0)
        wait_store(seq_len - 1, 1)

    return body(table2, x_t)


def kernel(x, embedding_table, train):
    b, s = x.shape
    table2 = _pack(embedding_table.T)
    out_t = _gather_t(table2, x.T, s, b)
    return out_t.transpose(2, 0, 1)


# speedup vs baseline: 2.0173x; 1.0625x over previous
import functools

import jax
import jax.numpy as jnp
from jax import lax
from jax.experimental import pallas as pl
from jax.experimental.pallas import tpu as pltpu
from jax.experimental.pallas import tpu_sc as plsc

D = 64
NC, NS = 2, 16
NW = NC * NS
BLK = 128
L = 16


_PACK_BW = 8192
_PACK_LOG = _PACK_BW.bit_length() - 1


def _pack_block(t_ref, o_ref):
    t = t_ref[...]
    o_ref[:, :D] = t[:, :_PACK_BW].T
    o_ref[:, D:] = t[:, _PACK_BW:].T


@jax.jit
def _pack(table_t):
    dd, v = table_t.shape
    grid = (v + 2 * _PACK_BW - 1) // (2 * _PACK_BW)
    return pl.pallas_call(
        _pack_block,
        grid=(grid,),
        in_specs=[pl.BlockSpec((dd, 2 * _PACK_BW), lambda i: (0, i))],
        out_specs=pl.BlockSpec((_PACK_BW, 2 * dd), lambda i: (i, 0)),
        out_shape=jax.ShapeDtypeStruct((grid * _PACK_BW, 2 * dd), jnp.float32),
    )(table_t)


@functools.partial(jax.jit, static_argnums=(2, 3))
def _gather_t(table2, x_t, seq_len, batch):
    assert batch // NW == BLK

    mesh = plsc.VectorSubcoreMesh(
        core_axis_name="c", subcore_axis_name="s",
        num_cores=NC, num_subcores=NS)

    @functools.partial(
        pl.kernel,
        mesh=mesh,
        compiler_params=pltpu.CompilerParams(
            use_tc_tiling_on_sc=True, needs_layout_passes=False),
        out_type=jax.ShapeDtypeStruct((seq_len, D, batch), jnp.float32),
        scratch_types=[
            pltpu.VMEM((seq_len, BLK), jnp.int32),
            pltpu.VMEM((seq_len, BLK), jnp.int32),
            pltpu.VMEM((2, BLK, BLK), jnp.float32),
            pltpu.VMEM((2, D, BLK), jnp.float32),
            pltpu.SemaphoreType.DMA,
            pltpu.SemaphoreType.DMA,
        ],
    )
    def body(table_hbm, xt_hbm, out_hbm, idx_v, idx2_v, rows_v, trans_v, gsem, osem):
        wid = lax.axis_index("s") * NC + lax.axis_index("c")
        col0 = wid * BLK

        pltpu.sync_copy(xt_hbm.at[:, pl.ds(col0, BLK)], idx_v)

        @plsc.parallel_loop(0, seq_len, unroll=4)
        def _(si):
            for g in range(BLK // L):
                v = idx_v[si, pl.ds(g * L, L)]
                idx2_v[si, pl.ds(g * L, L)] = lax.bitwise_or(
                    lax.shift_left(
                        lax.shift_right_logical(
                            v, jnp.int32(_PACK_LOG + 1)),
                        jnp.int32(_PACK_LOG)),
                    lax.bitwise_and(v, jnp.int32(_PACK_BW - 1)))

        def fire_gather(s, b):
            pltpu.async_copy(
                table_hbm.at[idx2_v.at[s]], rows_v.at[b], gsem)

        def wait_gather(s, b):
            pltpu.make_async_copy(
                table_hbm.at[idx2_v.at[s]], rows_v.at[b], gsem).wait()

        def fire_store(s, b):
            pltpu.async_copy(
                trans_v.at[b], out_hbm.at[s, :, pl.ds(col0, BLK)], osem)

        def wait_store(s, b):
            pltpu.make_async_copy(
                trans_v.at[b], out_hbm.at[s, :, pl.ds(col0, BLK)], osem).wait()

        riota = lax.iota(jnp.int32, L)
        perms = [lax.rem(riota + jnp.int32(k), jnp.int32(L)) for k in range(L)]

        def transpose_block(s, b):
            @plsc.parallel_loop(0, (D // L) * (BLK // L), unroll=2)
            def _(t):
                di = lax.div(t, jnp.int32(BLK // L)) * L
                ri = lax.rem(t, jnp.int32(BLK // L)) * L
                rvec = riota + ri
                pvec = lax.shift_left(
                    lax.bitwise_and(
                        lax.shift_right_logical(
                            plsc.load_gather(idx_v.at[s], [rvec]),
                            jnp.int32(_PACK_LOG)),
                        jnp.int32(1)),
                    jnp.int32(6))
                dvec0 = pvec + di
                svec0 = riota * jnp.int32(0) + di
                for k in range(L):
                    cvec = perms[k] + dvec0
                    v = plsc.load_gather(rows_v.at[b], [rvec, cvec])
                    plsc.store_scatter(
                        trans_v.at[b], [perms[k] + svec0, rvec], v)

        fire_gather(0, 0)

        @pl.loop(0, seq_len, step=2)
        def _(s2):
            for b in range(2):
                s = s2 + b
                nb = 1 - b

                @pl.when(s + 1 < seq_len)
                def _():
                    fire_gather(s + 1, nb)

                wait_gather(s, b)

                @pl.when(s >= 2)
                def _():
                    wait_store(s - 2, b)

                transpose_block(s, b)
                fire_store(s, b)

        wait_store(seq_len - 2, 0)
        wait_store(seq_len - 1, 1)

    return body(table2, x_t)


def kernel(x, embedding_table, train):
    b, s = x.shape
    table2 = _pack(embedding_table.T)
    out_t = _gather_t(table2, x.T, s, b)
    return out_t.transpose(2, 0, 1)


# speedup vs baseline: 2.0767x; 1.0294x over previous
import functools

import jax
import jax.numpy as jnp
from jax import lax
from jax.experimental import pallas as pl
from jax.experimental.pallas import tpu as pltpu
from jax.experimental.pallas import tpu_sc as plsc

D = 64
NC, NS = 2, 16
NW = NC * NS
BLK = 128
L = 16


_PACK_BW = 16384
_PACK_LOG = _PACK_BW.bit_length() - 1


def _pack_block(t_ref, o_ref):
    t = t_ref[...]
    o_ref[:, :D] = t[:, :_PACK_BW].T
    o_ref[:, D:] = t[:, _PACK_BW:].T


@jax.jit
def _pack(table_t):
    dd, v = table_t.shape
    grid = (v + 2 * _PACK_BW - 1) // (2 * _PACK_BW)
    return pl.pallas_call(
        _pack_block,
        grid=(grid,),
        in_specs=[pl.BlockSpec((dd, 2 * _PACK_BW), lambda i: (0, i))],
        out_specs=pl.BlockSpec((_PACK_BW, 2 * dd), lambda i: (i, 0)),
        out_shape=jax.ShapeDtypeStruct((grid * _PACK_BW, 2 * dd), jnp.float32),
    )(table_t)


@functools.partial(jax.jit, static_argnums=(2, 3))
def _gather_t(table2, x_t, seq_len, batch):
    assert batch // NW == BLK

    mesh = plsc.VectorSubcoreMesh(
        core_axis_name="c", subcore_axis_name="s",
        num_cores=NC, num_subcores=NS)

    @functools.partial(
        pl.kernel,
        mesh=mesh,
        compiler_params=pltpu.CompilerParams(
            use_tc_tiling_on_sc=True, needs_layout_passes=False),
        out_type=jax.ShapeDtypeStruct((seq_len, D, batch), jnp.float32),
        scratch_types=[
            pltpu.VMEM((seq_len, BLK), jnp.int32),
            pltpu.VMEM((seq_len, BLK), jnp.int32),
            pltpu.VMEM((2, BLK, BLK), jnp.float32),
            pltpu.VMEM((2, D, BLK), jnp.float32),
            pltpu.SemaphoreType.DMA,
            pltpu.SemaphoreType.DMA,
        ],
    )
    def body(table_hbm, xt_hbm, out_hbm, idx_v, idx2_v, rows_v, trans_v, gsem, osem):
        wid = lax.axis_index("s") * NC + lax.axis_index("c")
        col0 = wid * BLK

        pltpu.sync_copy(xt_hbm.at[:, pl.ds(col0, BLK)], idx_v)

        @plsc.parallel_loop(0, seq_len, unroll=4)
        def _(si):
            for g in range(BLK // L):
                v = idx_v[si, pl.ds(g * L, L)]
                idx2_v[si, pl.ds(g * L, L)] = lax.bitwise_or(
                    lax.shift_left(
                        lax.shift_right_logical(
                            v, jnp.int32(_PACK_LOG + 1)),
                        jnp.int32(_PACK_LOG)),
                    lax.bitwise_and(v, jnp.int32(_PACK_BW - 1)))

        def fire_gather(s, b):
            pltpu.async_copy(
                table_hbm.at[idx2_v.at[s]], rows_v.at[b], gsem)

        def wait_gather(s, b):
            pltpu.make_async_copy(
                table_hbm.at[idx2_v.at[s]], rows_v.at[b], gsem).wait()

        def fire_store(s, b):
            pltpu.async_copy(
                trans_v.at[b], out_hbm.at[s, :, pl.ds(col0, BLK)], osem)

        def wait_store(s, b):
            pltpu.make_async_copy(
                trans_v.at[b], out_hbm.at[s, :, pl.ds(col0, BLK)], osem).wait()

        riota = lax.iota(jnp.int32, L)
        perms = [lax.rem(riota + jnp.int32(k), jnp.int32(L)) for k in range(L)]

        def transpose_block(s, b):
            @plsc.parallel_loop(0, (D // L) * (BLK // L), unroll=2)
            def _(t):
                di = lax.div(t, jnp.int32(BLK // L)) * L
                ri = lax.rem(t, jnp.int32(BLK // L)) * L
                rvec = riota + ri
                pvec = lax.shift_left(
                    lax.bitwise_and(
                        lax.shift_right_logical(
                            plsc.load_gather(idx_v.at[s], [rvec]),
                            jnp.int32(_PACK_LOG)),
                        jnp.int32(1)),
                    jnp.int32(6))
                dvec0 = pvec + di
                svec0 = riota * jnp.int32(0) + di
                for k in range(L):
                    cvec = perms[k] + dvec0
                    v = plsc.load_gather(rows_v.at[b], [rvec, cvec])
                    plsc.store_scatter(
                        trans_v.at[b], [perms[k] + svec0, rvec], v)

        fire_gather(0, 0)

        @pl.loop(0, seq_len, step=2)
        def _(s2):
            for b in range(2):
                s = s2 + b
                nb = 1 - b

                @pl.when(s + 1 < seq_len)
                def _():
                    fire_gather(s + 1, nb)

                wait_gather(s, b)

                @pl.when(s >= 2)
                def _():
                    wait_store(s - 2, b)

                transpose_block(s, b)
                fire_store(s, b)

        wait_store(seq_len - 2, 0)
        wait_store(seq_len - 1, 1)

    return body(table2, x_t)


def kernel(x, embedding_table, train):
    b, s = x.shape
    table2 = _pack(embedding_table.T)
    out_t = _gather_t(table2, x.T, s, b)
    return out_t.transpose(2, 0, 1)
